# Initial kernel scaffold; baseline (speedup 1.0000x reference)
import jax
import jax.numpy as jnp
from jax.experimental import pallas as pl


def kernel(image, displacement_field):
    raise NotImplementedError("write your pallas kernel here")


# speedup vs baseline: 3.4344x; 3.4344x over previous
import jax
import jax.numpy as jnp
from jax import lax
from jax.experimental import pallas as pl
from jax.experimental.pallas import tpu as pltpu
from jax.experimental.pallas import tpu_sc as plsc

B, C, D, H, W = 2, 8, 96, 96, 96
DHW = D * H * W
NW = 32
VPT = (B * DHW) // NW
VC = 128
NCHUNK = VPT // VC
NG = VC // 16


def _sc_body(table, dx, dy, dz, rows, *scr):
    Is = scr[0:4]
    Us = scr[4:12]
    Gs = scr[12:16]
    Ds = scr[16:19]
    Os = scr[19]
    sem = scr[20]
    wid = lax.axis_index("s") * 2 + lax.axis_index("c")
    b = wid // (NW // B)
    boff = b * DHW
    lane = lax.iota(jnp.int32, 16)
    lmask = lane < 8

    def chunk(i, carry):
        gbase = wid * VPT + i * VC
        vloc = gbase - boff
        for a, src in enumerate((dx, dy, dz)):
            pltpu.sync_copy(src.at[pl.ds(gbase, VC)], Ds[a])

        def grp(g, c2):
            s = g * 16
            vv = vloc + s + lane
            d_ = lax.div(vv, H * W)
            r_ = vv - d_ * (H * W)
            h_ = lax.div(r_, W)
            w_ = r_ - h_ * W

            def axis_calc(ivec, dref, lo):
                co = ivec.astype(jnp.float32) + dref[pl.ds(s, 16)]
                ic = ((co + 1.0) * 96.0 - 1.0) / 2.0
                ic = jnp.minimum(jnp.maximum(ic, -2.0), 97.0)
                it = ic.astype(jnp.int32)
                ft = it.astype(jnp.float32)
                adj = ft > ic
                i0 = it - jnp.where(adj, 1, 0)
                f0 = ft - jnp.where(adj, 1.0, 0.0)
                t = ic - f0
                i1 = i0 + 1
                m0 = jnp.where((i0 >= 0) & (i0 < 96), 1.0, 0.0)
                m1 = jnp.where((i1 >= 0) & (i1 < 96), 1.0, 0.0)
                a0 = (1.0 - t) * m0
                a1 = t * m1
                c0 = jnp.minimum(jnp.maximum(i0, lo), 95)
                return c0, a0, a1

            xc0, ax0, ax1 = axis_calc(d_, Ds[0], -1)
            yc0, ay0, ay1 = axis_calc(h_, Ds[1], 0)
            zc0, az0, az1 = axis_calc(w_, Ds[2], 0)
            yc1 = jnp.minimum(yc0 + 1, 95)
            zc1 = jnp.minimum(zc0 + 1, 95)
            base = xc0 + 1 + boff
            p = 0
            for zc, az in ((zc0, az0), (zc1, az1)):
                for yc, ay in ((yc0, ay0), (yc1, ay1)):
                    Is[p][pl.ds(s, 16)] = (zc * 96 + yc) * 96 + base
                    u = az * ay
                    Us[2 * p][pl.ds(s, 16)] = u * ax0
                    Us[2 * p + 1][pl.ds(s, 16)] = u * ax1
                    p += 1
            return c2

        lax.fori_loop(0, NG, grp, 0, unroll=False)

        cps = [pltpu.async_copy(table.at[Is[p]], Gs[p], sem) for p in range(4)]
        for cp in cps:
            cp.wait()

        def grp2(g, c2):
            s = g * 16
            uvv = [Us[k][pl.ds(s, 16)] for k in range(8)]
            for j in range(16):
                v = s + j
                acc = None
                for p in range(4):
                    row = Gs[p][v, :]
                    wv = jnp.where(lmask, uvv[2 * p][j], uvv[2 * p + 1][j])
                    acc = row * wv if acc is None else acc + row * wv
                Os[v, :] = acc
            return c2

        lax.fori_loop(0, NG, grp2, 0, unroll=False)
        pltpu.sync_copy(Os, rows.at[pl.ds(gbase, VC), :])
        return carry

    lax.fori_loop(0, NCHUNK, chunk, 0, unroll=False)


_mesh = plsc.VectorSubcoreMesh(core_axis_name="c", subcore_axis_name="s")
_scratch = (
    [pltpu.VMEM((VC,), jnp.int32) for _ in range(4)]
    + [pltpu.VMEM((VC,), jnp.float32) for _ in range(8)]
    + [pltpu.VMEM((VC, 16), jnp.float32) for _ in range(4)]
    + [pltpu.VMEM((VC,), jnp.float32) for _ in range(3)]
    + [pltpu.VMEM((VC, 16), jnp.float32)]
    + [pltpu.SemaphoreType.DMA]
)

_sc_sample = pl.kernel(
    _sc_body,
    out_type=jax.ShapeDtypeStruct((B * DHW, 16), jnp.float32),
    mesh=_mesh,
    scratch_types=_scratch,
    compiler_params=pltpu.CompilerParams(use_tc_tiling_on_sc=False),
)

VB = 8192
NB = DHW // VB


def _tc_body(rows_ref, out_ref):
    r = jnp.arange(16, dtype=jnp.int32)[:, None]
    c = jnp.arange(C, dtype=jnp.int32)[None, :]
    sel = ((r == c) | (r == c + 8)).astype(jnp.float32)
    out_ref[:, :] = lax.dot_general(
        sel, rows_ref[:, :], (((0,), (1,)), ((), ())),
        preferred_element_type=jnp.float32,
    )


_tc_fold = pl.pallas_call(
    _tc_body,
    grid=(B, NB),
    in_specs=[pl.BlockSpec((VB, 16), lambda b, j: (b * NB + j, 0))],
    out_specs=pl.BlockSpec((C, VB), lambda b, j: (b, j)),
    out_shape=jax.ShapeDtypeStruct((B * C, DHW), jnp.float32),
)


def kernel(image, displacement_field):
    Tg = image.transpose(0, 2, 3, 4, 1).reshape(B * DHW, C)
    z8 = jnp.zeros((1, C), jnp.float32)
    P = jnp.concatenate(
        [jnp.concatenate([z8, Tg], 0), jnp.concatenate([Tg, z8], 0)], axis=1
    )
    disp = jnp.moveaxis(displacement_field, -1, 0).reshape(3, B * DHW)
    rows = _sc_sample(P, disp[0], disp[1], disp[2])
    out = _tc_fold(rows)
    return out.reshape(B, C, D, H, W)


# speedup vs baseline: 3.4375x; 1.0009x over previous
import jax
import jax.numpy as jnp
from jax import lax
from jax.experimental import pallas as pl
from jax.experimental.pallas import tpu as pltpu
from jax.experimental.pallas import tpu_sc as plsc

B, C, D, H, W = 2, 8, 96, 96, 96
DHW = D * H * W
NW = 32
VPT = (B * DHW) // NW
VC = 128
NCHUNK = VPT // VC
NG = VC // 16


def _sc_body(table, dx, dy, dz, rows, *scr):
    IsA, IsB = scr[0:4], scr[4:8]
    UsA, UsB = scr[8:16], scr[16:24]
    GsA, GsB = scr[24:28], scr[28:32]
    DsA, DsB = scr[32:35], scr[35:38]
    OsA, OsB = scr[38], scr[39]
    semdA, semdB, semgA, semgB, semoA, semoB = scr[40:46]
    wid = lax.axis_index("s") * 2 + lax.axis_index("c")
    b = wid // (NW // B)
    boff = b * DHW
    lane = lax.iota(jnp.int32, 16)
    lmask = lane < 8
    dsrcs = (dx, dy, dz)

    def disp_prefetch(i, Dsb, sem):
        gbase = wid * VPT + i * VC
        for a in range(3):
            pltpu.async_copy(dsrcs[a].at[pl.ds(gbase, VC)], Dsb[a], sem)

    def disp_wait(i, Dsb, sem):
        gbase = wid * VPT + i * VC
        for a in range(3):
            pltpu.make_async_copy(dsrcs[a].at[pl.ds(gbase, VC)], Dsb[a], sem).wait()

    def phase_a(i, Dsb, Isb, Usb):
        vloc = wid * VPT + i * VC - boff

        def grp(g, c2):
            s = g * 16
            vv = vloc + s + lane
            d_ = lax.div(vv, H * W)
            r_ = vv - d_ * (H * W)
            h_ = lax.div(r_, W)
            w_ = r_ - h_ * W

            def axis_calc(ivec, dref, lo):
                co = ivec.astype(jnp.float32) + dref[pl.ds(s, 16)]
                ic = ((co + 1.0) * 96.0 - 1.0) / 2.0
                ic = jnp.minimum(jnp.maximum(ic, -2.0), 97.0)
                it = ic.astype(jnp.int32)
                ft = it.astype(jnp.float32)
                adj = ft > ic
                i0 = it - jnp.where(adj, 1, 0)
                f0 = ft - jnp.where(adj, 1.0, 0.0)
                t = ic - f0
                i1 = i0 + 1
                m0 = jnp.where((i0 >= 0) & (i0 < 96), 1.0, 0.0)
                m1 = jnp.where((i1 >= 0) & (i1 < 96), 1.0, 0.0)
                a0 = (1.0 - t) * m0
                a1 = t * m1
                c0 = jnp.minimum(jnp.maximum(i0, lo), 95)
                return c0, a0, a1

            xc0, ax0, ax1 = axis_calc(d_, Dsb[0], -1)
            yc0, ay0, ay1 = axis_calc(h_, Dsb[1], 0)
            zc0, az0, az1 = axis_calc(w_, Dsb[2], 0)
            yc1 = jnp.minimum(yc0 + 1, 95)
            zc1 = jnp.minimum(zc0 + 1, 95)
            base = xc0 + 1 + boff
            p = 0
            for zc, az in ((zc0, az0), (zc1, az1)):
                for yc, ay in ((yc0, ay0), (yc1, ay1)):
                    Isb[p][pl.ds(s, 16)] = (zc * 96 + yc) * 96 + base
                    u = az * ay
                    Usb[2 * p][pl.ds(s, 16)] = u * ax0
                    Usb[2 * p + 1][pl.ds(s, 16)] = u * ax1
                    p += 1
            return c2

        lax.fori_loop(0, NG, grp, 0, unroll=False)

    def fire_gathers(Isb, Gsb, sem):
        for p in range(4):
            pltpu.async_copy(table.at[Isb[p]], Gsb[p], sem)

    def wait_gathers(Isb, Gsb, sem):
        for p in range(4):
            pltpu.make_async_copy(table.at[Isb[p]], Gsb[p], sem).wait()

    def combine(Gsb, Usb, Osb):
        def grp2(g, c2):
            s = g * 16
            uvv = [Usb[k][pl.ds(s, 16)] for k in range(8)]
            for j in range(16):
                v = s + j
                acc = None
                for p in range(4):
                    row = Gsb[p][v, :]
                    wv = jnp.where(lmask, uvv[2 * p][j], uvv[2 * p + 1][j])
                    acc = row * wv if acc is None else acc + row * wv
                Osb[v, :] = acc
            return c2

        lax.fori_loop(0, NG, grp2, 0, unroll=False)

    def fire_out(i, Osb, sem):
        gbase = wid * VPT + i * VC
        pltpu.async_copy(Osb, rows.at[pl.ds(gbase, VC), :], sem)

    def wait_out(i, Osb, sem):
        gbase = wid * VPT + i * VC
        pltpu.make_async_copy(Osb, rows.at[pl.ds(gbase, VC), :], sem).wait()

    disp_prefetch(0, DsA, semdA)

    def body(k, carry):
        i0 = 2 * k
        i1 = 2 * k + 1
        disp_wait(i0, DsA, semdA)
        phase_a(i0, DsA, IsA, UsA)
        fire_gathers(IsA, GsA, semgA)
        disp_prefetch(i1, DsB, semdB)

        @pl.when(k > 0)
        def _():
            wait_gathers(IsB, GsB, semgB)

            @pl.when(k > 1)
            def _():
                wait_out(i0 - 3, OsB, semoB)

            combine(GsB, UsB, OsB)
            fire_out(i0 - 1, OsB, semoB)

        disp_wait(i1, DsB, semdB)
        phase_a(i1, DsB, IsB, UsB)
        fire_gathers(IsB, GsB, semgB)

        @pl.when(i1 + 1 < NCHUNK)
        def _():
            disp_prefetch(i1 + 1, DsA, semdA)

        wait_gathers(IsA, GsA, semgA)

        @pl.when(k > 0)
        def _():
            wait_out(i0 - 2, OsA, semoA)

        combine(GsA, UsA, OsA)
        fire_out(i0, OsA, semoA)
        return carry

    lax.fori_loop(0, NCHUNK // 2, body, 0, unroll=False)
    wait_gathers(IsB, GsB, semgB)
    wait_out(NCHUNK - 3, OsB, semoB)
    combine(GsB, UsB, OsB)
    fire_out(NCHUNK - 1, OsB, semoB)
    wait_out(NCHUNK - 2, OsA, semoA)
    wait_out(NCHUNK - 1, OsB, semoB)


_mesh = plsc.VectorSubcoreMesh(core_axis_name="c", subcore_axis_name="s")
_scratch = (
    [pltpu.VMEM((VC,), jnp.int32) for _ in range(8)]
    + [pltpu.VMEM((VC,), jnp.float32) for _ in range(16)]
    + [pltpu.VMEM((VC, 16), jnp.float32) for _ in range(8)]
    + [pltpu.VMEM((VC,), jnp.float32) for _ in range(6)]
    + [pltpu.VMEM((VC, 16), jnp.float32) for _ in range(2)]
    + [pltpu.SemaphoreType.DMA for _ in range(6)]
)

_sc_sample = pl.kernel(
    _sc_body,
    out_type=jax.ShapeDtypeStruct((B * DHW, 16), jnp.float32),
    mesh=_mesh,
    scratch_types=_scratch,
    compiler_params=pltpu.CompilerParams(use_tc_tiling_on_sc=False),
)

VB = 8192
NB = DHW // VB


def _tc_body(rows_ref, out_ref):
    r = jnp.arange(16, dtype=jnp.int32)[:, None]
    c = jnp.arange(C, dtype=jnp.int32)[None, :]
    sel = ((r == c) | (r == c + 8)).astype(jnp.float32)
    out_ref[:, :] = lax.dot_general(
        sel, rows_ref[:, :], (((0,), (1,)), ((), ())),
        preferred_element_type=jnp.float32,
    )


_tc_fold = pl.pallas_call(
    _tc_body,
    grid=(B, NB),
    in_specs=[pl.BlockSpec((VB, 16), lambda b, j: (b * NB + j, 0))],
    out_specs=pl.BlockSpec((C, VB), lambda b, j: (b, j)),
    out_shape=jax.ShapeDtypeStruct((B * C, DHW), jnp.float32),
)


def kernel(image, displacement_field):
    Tg = image.transpose(0, 2, 3, 4, 1).reshape(B * DHW, C)
    z8 = jnp.zeros((1, C), jnp.float32)
    P = jnp.concatenate(
        [jnp.concatenate([z8, Tg], 0), jnp.concatenate([Tg, z8], 0)], axis=1
    )
    disp = jnp.moveaxis(displacement_field, -1, 0).reshape(3, B * DHW)
    rows = _sc_sample(P, disp[0], disp[1], disp[2])
    out = _tc_fold(rows)
    return out.reshape(B, C, D, H, W)
